# baseline (speedup 1.0000x reference)
import dataclasses
import functools

import jax
import jax.numpy as jnp
from jax import lax
from jax.experimental import pallas as pl
from jax.experimental.pallas import tpu as pltpu
from jax.experimental.pallas import tpu_sc as plsc

H = 480
W = 640
RY = 8
GH = H + 2 * RY
PLANE = GH * W
ACC = 2 * PLANE
INT_OFF = RY * W
OUT_B = 2 * H * W
NTILES = 16
BPC = 4
CHUNK = 1792
NVEC = CHUNK // 16
ENTRIES = 4 * CHUNK
EV = 6 * CHUNK
ZS = ACC // NTILES
ZTAIL = ZS - (ZS // ENTRIES) * ENTRIES
WS = OUT_B // NTILES


def _splat(pk, batches, nch):
    chunks_per_tile = nch // NTILES
    mesh = plsc.VectorSubcoreMesh(core_axis_name="c", subcore_axis_name="s")
    cp = pltpu.CompilerParams()
    if "needs_layout_passes" in pltpu.CompilerParams.__dataclass_fields__:
        cp = dataclasses.replace(cp, needs_layout_passes=False)

    @functools.partial(
        pl.kernel,
        compiler_params=cp,
        out_type=jax.ShapeDtypeStruct((batches * OUT_B,), jnp.float32),
        mesh=mesh,
        scratch_types=[
            pltpu.VMEM((EV,), jnp.float32),
            pltpu.VMEM((EV,), jnp.float32),
            pltpu.VMEM((ENTRIES,), jnp.int32),
            pltpu.VMEM((ENTRIES,), jnp.int32),
            pltpu.VMEM((ENTRIES,), jnp.float32),
            pltpu.VMEM((ENTRIES,), jnp.float32),
            pltpu.VMEM_SHARED((ACC,), jnp.float32),
            pltpu.SemaphoreType.DMA,
            pltpu.SemaphoreType.DMA,
            pltpu.SemaphoreType.DMA,
            pltpu.SemaphoreType.DMA,
        ],
    )
    def k(pk_hbm, out_hbm, ev0, ev1, idx0, idx1, val0, val1, acc_sh,
          sin0, sin1, ssc0, ssc1):
        c = lax.axis_index("c")
        s = lax.axis_index("s")
        evs = (ev0, ev1)
        idxs = (idx0, idx1)
        vals = (val0, val1)
        sins = (sin0, sin1)
        sscs = (ssc0, ssc1)
        n_pad = nch * CHUNK

        def compute_chunk(ev_v, idx_v, val_v):
            @pl.loop(0, NVEC)
            def _(vi):
                o16 = vi * 16
                ts = ev_v[pl.ds(o16, 16)]
                ly = ev_v[pl.ds(CHUNK + o16, 16)]
                lx = ev_v[pl.ds(2 * CHUNK + o16, 16)]
                fy = ev_v[pl.ds(3 * CHUNK + o16, 16)]
                fx = ev_v[pl.ds(4 * CHUNK + o16, 16)]
                po = ev_v[pl.ds(5 * CHUNK + o16, 16)]
                t = 1.0 - ts
                wy = ly + t * fy
                wx = lx + t * fx
                yi = (wy + 512.0).astype(jnp.int32)
                dy = wy - (yi.astype(jnp.float32) - 512.0)
                xi = (wx + 512.0).astype(jnp.int32)
                dx = wx - (xi.astype(jnp.float32) - 512.0)
                yg = jnp.minimum(jnp.maximum(yi - (512 - RY), 0), GH - 2)
                x0 = xi - 512
                m0 = (x0 >= 0) & (x0 <= W - 1)
                m1 = (x0 >= -1) & (x0 <= W - 2)
                cx0 = jnp.minimum(jnp.maximum(x0, 0), W - 1)
                cx1 = jnp.minimum(jnp.maximum(x0 + 1, 0), W - 1)
                pz = po.astype(jnp.int32)
                rb = (1 - pz) * PLANE + yg * W
                uy = 1.0 - dy
                ux = 1.0 - dx
                zv = jnp.zeros((16,), jnp.float32)
                o = vi * 64
                idx_v[pl.ds(o, 16)] = rb + cx0
                val_v[pl.ds(o, 16)] = jnp.where(m0, uy * ux, zv)
                idx_v[pl.ds(o + 16, 16)] = rb + cx1
                val_v[pl.ds(o + 16, 16)] = jnp.where(m1, uy * dx, zv)
                idx_v[pl.ds(o + 32, 16)] = rb + W + cx0
                val_v[pl.ds(o + 32, 16)] = jnp.where(m0, dy * ux, zv)
                idx_v[pl.ds(o + 48, 16)] = rb + W + cx1
                val_v[pl.ds(o + 48, 16)] = jnp.where(m1, dy * dx, zv)

        @pl.loop(0, BPC)
        def _(bi):
            b = c * BPC + bi

            @pl.loop(0, ENTRIES // 16)
            def _(i):
                val0[pl.ds(i * 16, 16)] = jnp.zeros((16,), jnp.float32)

            @pl.loop(0, ZS // ENTRIES)
            def _(zi):
                pltpu.sync_copy(
                    val0, acc_sh.at[pl.ds(s * ZS + zi * ENTRIES, ENTRIES)])
            pltpu.sync_copy(
                val0.at[pl.ds(0, ZTAIL)],
                acc_sh.at[pl.ds(s * ZS + (ZS // ENTRIES) * ENTRIES, ZTAIL)])
            plsc.subcore_barrier()

            def start_in(kk):
                off = (s + kk * NTILES) * CHUNK
                row0 = b * 6 * n_pad + off
                pb = kk % 2
                return [
                    pltpu.async_copy(
                        pk_hbm.at[pl.ds(row0 + r * n_pad, CHUNK)],
                        evs[pb].at[pl.ds(r * CHUNK, CHUNK)], sins[pb])
                    for r in range(6)
                ]

            hin = {0: start_in(0)}
            hsc = [None, None]
            for kk in range(chunks_per_tile):
                pb = kk % 2
                if kk + 1 < chunks_per_tile:
                    hin[kk + 1] = start_in(kk + 1)
                for h in hin.pop(kk):
                    h.wait()
                if hsc[pb] is not None:
                    hsc[pb].wait()
                compute_chunk(evs[pb], idxs[pb], vals[pb])
                hsc[pb] = pltpu.async_copy(
                    vals[pb], acc_sh.at[idxs[pb]], sscs[pb], add=True)
            for h in hsc:
                if h is not None:
                    h.wait()

            plsc.subcore_barrier()
            acc_off = (s // 8) * PLANE + INT_OFF + (s % 8) * WS
            pltpu.sync_copy(
                acc_sh.at[pl.ds(acc_off, WS)],
                out_hbm.at[pl.ds(b * OUT_B + s * WS, WS)])
            plsc.subcore_barrier()

    return k(pk)


def kernel(event_ts, event_loc, event_flow, pol_mask):
    B, N, _ = event_ts.shape
    step = CHUNK * NTILES
    n_pad = ((N + step - 1) // step) * step
    nch = n_pad // CHUNK
    ts = event_ts[..., 0]
    po = pol_mask[..., 0]
    pk = jnp.stack([ts, event_loc[..., 0], event_loc[..., 1],
                    event_flow[..., 0], event_flow[..., 1], po], axis=1)
    if n_pad > N:
        padblk = jnp.zeros((B, 6, n_pad - N), jnp.float32).at[:, 2, :].set(-1e6)
        pk = jnp.concatenate([pk, padblk], axis=2)
    out = _splat(pk.reshape(-1), B, nch)
    return out.reshape(B, 2, H, W)


# speedup vs baseline: 1.1242x; 1.1242x over previous
import dataclasses
import functools

import jax
import jax.numpy as jnp
from jax import lax
from jax.experimental import pallas as pl
from jax.experimental.pallas import tpu as pltpu
from jax.experimental.pallas import tpu_sc as plsc

H = 480
W = 640
RY = 8
GH = H + 2 * RY
PLANE = GH * W
ACC = 2 * PLANE
INT_OFF = RY * W
OUT_B = 2 * H * W
NTILES = 16
BPC = 4
CHUNK = 1792
NVEC = CHUNK // 16
ENTRIES = 4 * CHUNK
EV = 6 * CHUNK
ZS = ACC // NTILES
ZTAIL = ZS - (ZS // ENTRIES) * ENTRIES
WS = OUT_B // NTILES


def _splat(pk, batches, nch):
    chunks_per_tile = nch // NTILES
    mesh = plsc.VectorSubcoreMesh(core_axis_name="c", subcore_axis_name="s")
    cp = pltpu.CompilerParams()
    if "needs_layout_passes" in pltpu.CompilerParams.__dataclass_fields__:
        cp = dataclasses.replace(cp, needs_layout_passes=False)

    @functools.partial(
        pl.kernel,
        compiler_params=cp,
        out_type=jax.ShapeDtypeStruct((batches * OUT_B,), jnp.float32),
        mesh=mesh,
        scratch_types=[
            pltpu.VMEM((EV,), jnp.float32),
            pltpu.VMEM((EV,), jnp.float32),
            pltpu.VMEM((ENTRIES,), jnp.int32),
            pltpu.VMEM((ENTRIES,), jnp.int32),
            pltpu.VMEM((ENTRIES,), jnp.float32),
            pltpu.VMEM((ENTRIES,), jnp.float32),
            pltpu.VMEM_SHARED((ACC,), jnp.float32),
            pltpu.SemaphoreType.DMA,
            pltpu.SemaphoreType.DMA,
            pltpu.SemaphoreType.DMA,
            pltpu.SemaphoreType.DMA,
        ],
    )
    def k(pk_hbm, out_hbm, ev0, ev1, idx0, idx1, val0, val1, acc_sh,
          sin0, sin1, ssc0, ssc1):
        c = lax.axis_index("c")
        s = lax.axis_index("s")
        evs = (ev0, ev1)
        idxs = (idx0, idx1)
        vals = (val0, val1)
        sins = (sin0, sin1)
        sscs = (ssc0, ssc1)
        n_pad = nch * CHUNK

        def compute_chunk(ev_v, idx_v, val_v):
            @pl.loop(0, NVEC)
            def _(vi):
                o16 = vi * 16
                ts = ev_v[pl.ds(o16, 16)]
                ly = ev_v[pl.ds(CHUNK + o16, 16)]
                lx = ev_v[pl.ds(2 * CHUNK + o16, 16)]
                fy = ev_v[pl.ds(3 * CHUNK + o16, 16)]
                fx = ev_v[pl.ds(4 * CHUNK + o16, 16)]
                po = ev_v[pl.ds(5 * CHUNK + o16, 16)]
                t = 1.0 - ts
                wy = ly + t * fy
                wx = lx + t * fx
                yi = (wy + 512.0).astype(jnp.int32)
                dy = wy - (yi.astype(jnp.float32) - 512.0)
                xi = (wx + 512.0).astype(jnp.int32)
                dx = wx - (xi.astype(jnp.float32) - 512.0)
                yg = jnp.minimum(jnp.maximum(yi - (512 - RY), 0), GH - 2)
                x0 = xi - 512
                m0 = (x0 >= 0) & (x0 <= W - 1)
                m1 = (x0 >= -1) & (x0 <= W - 2)
                cx0 = jnp.minimum(jnp.maximum(x0, 0), W - 1)
                cx1 = jnp.minimum(jnp.maximum(x0 + 1, 0), W - 1)
                pz = po.astype(jnp.int32)
                rb = (1 - pz) * PLANE + yg * W
                uy = 1.0 - dy
                ux = 1.0 - dx
                zv = jnp.zeros((16,), jnp.float32)
                o = vi * 64
                idx_v[pl.ds(o, 16)] = rb + cx0
                val_v[pl.ds(o, 16)] = jnp.where(m0, uy * ux, zv)
                idx_v[pl.ds(o + 16, 16)] = rb + cx1
                val_v[pl.ds(o + 16, 16)] = jnp.where(m1, uy * dx, zv)
                idx_v[pl.ds(o + 32, 16)] = rb + W + cx0
                val_v[pl.ds(o + 32, 16)] = jnp.where(m0, dy * ux, zv)
                idx_v[pl.ds(o + 48, 16)] = rb + W + cx1
                val_v[pl.ds(o + 48, 16)] = jnp.where(m1, dy * dx, zv)

        @pl.loop(0, BPC)
        def _(bi):
            b = c * BPC + bi

            @pl.loop(0, ENTRIES // 16)
            def _(i):
                val0[pl.ds(i * 16, 16)] = jnp.zeros((16,), jnp.float32)

            @pl.loop(0, ZS // ENTRIES)
            def _(zi):
                pltpu.sync_copy(
                    val0, acc_sh.at[pl.ds(s * ZS + zi * ENTRIES, ENTRIES)])
            pltpu.sync_copy(
                val0.at[pl.ds(0, ZTAIL)],
                acc_sh.at[pl.ds(s * ZS + (ZS // ENTRIES) * ENTRIES, ZTAIL)])
            plsc.subcore_barrier()

            def start_in(kk):
                off = (s + kk * NTILES) * CHUNK
                row0 = b * 6 * n_pad + off
                pb = kk % 2
                return [
                    pltpu.async_copy(
                        pk_hbm.at[pl.ds(row0 + r * n_pad, CHUNK)],
                        evs[pb].at[pl.ds(r * CHUNK, CHUNK)], sins[pb])
                    for r in range(6)
                ]

            hin = {0: start_in(0)}
            hsc = [None, None]
            for kk in range(chunks_per_tile):
                pb = kk % 2
                if kk + 1 < chunks_per_tile:
                    hin[kk + 1] = start_in(kk + 1)
                for h in hin.pop(kk):
                    h.wait()
                if hsc[pb] is not None:
                    hsc[pb].wait()
                compute_chunk(evs[pb], idxs[pb], vals[pb])
                hsc[pb] = pltpu.async_copy(
                    vals[pb], acc_sh.at[idxs[pb]], sscs[pb], add=True)
            for h in hsc:
                if h is not None:
                    h.wait()

            plsc.subcore_barrier()
            acc_off = (s // 8) * PLANE + INT_OFF + (s % 8) * WS
            pltpu.sync_copy(
                acc_sh.at[pl.ds(acc_off, WS)],
                out_hbm.at[pl.ds(b * OUT_B + s * WS, WS)])
            plsc.subcore_barrier()

    return k(pk)


def kernel(event_ts, event_loc, event_flow, pol_mask):
    B, N, _ = event_ts.shape
    step = CHUNK * NTILES
    n_pad = ((N + step - 1) // step) * step
    nch = n_pad // CHUNK
    def row(r, fill=0.0):
        if n_pad == N:
            return r
        return jnp.pad(r, ((0, 0), (0, n_pad - N)), constant_values=fill)

    pk = jnp.stack([row(event_ts[..., 0]), row(event_loc[..., 0]),
                    row(event_loc[..., 1], -1e6), row(event_flow[..., 0]),
                    row(event_flow[..., 1]), row(pol_mask[..., 0])], axis=1)
    out = _splat(pk.reshape(-1), B, nch)
    return out.reshape(B, 2, H, W)
